# Initial kernel scaffold; baseline (speedup 1.0000x reference)
import jax
import jax.numpy as jnp
from jax.experimental import pallas as pl


def kernel(dist, angle, idx_t, index_t, index_h):
    raise NotImplementedError("write your pallas kernel here")


# speedup vs baseline: 10.6999x; 10.6999x over previous
import functools

import jax
import jax.numpy as jnp
from jax import lax
from jax.experimental import pallas as pl
from jax.experimental.pallas import tpu as pltpu
from jax.experimental.pallas import tpu_sc as plsc

B, L, A = 4096, 32, 4
H, W = 50000, 16
T = L + 1
NCLS = 22
TOT = H * W
NW = 32
PER_W = 25600
PAD = NW * PER_W
CH = 3200
ROWS_C = 200
MM_ROWS = 320


def _tables_body(dist_ref, ang_ref, dtab_ref, atab_ref):
    bsz = dist_ref.shape[0]
    d = dist_ref[...]
    dtab_ref[...] = jnp.concatenate(
        [d, jnp.zeros((bsz, 1), jnp.float32)], axis=1)
    a = ang_ref[...]
    sc = jnp.concatenate([jnp.sin(a), jnp.cos(a)], axis=2)
    atab_ref[...] = jnp.concatenate(
        [sc, jnp.zeros((bsz, 1, 8), jnp.float32)], axis=1)


def _flatidx_body(ih_ref, it_ref, out_ref):
    out_ref[...] = ih_ref[...] * T + it_ref[...]


def _gather_body(dtab, atab, fidx, dout, aout, idx_v, d_v, a_v, s1, s2):
    wid = lax.axis_index("s") * 2 + lax.axis_index("c")
    base = wid * PER_W

    def chunk(k, carry):
        off = base + k * CH
        pltpu.sync_copy(fidx.at[pl.ds(off, CH)], idx_v)
        cp1 = pltpu.async_copy(dtab.at[idx_v], d_v, s1)
        cp2 = pltpu.async_copy(atab.at[idx_v], a_v, s2)
        cp1.wait()
        cp2.wait()
        pltpu.sync_copy(d_v, dout.at[pl.ds(off, CH)])
        pltpu.sync_copy(a_v, aout.at[pl.ds(off, CH)])
        return carry

    lax.fori_loop(0, PER_W // CH, chunk, 0)


def _minmax_body(x_ref, mn_ref, mx_ref, acc_mn, acc_mx):
    i = pl.program_id(0)

    @pl.when(i == 0)
    def _init():
        acc_mn[...] = jnp.full((MM_ROWS, 128), jnp.inf, jnp.float32)
        acc_mx[...] = jnp.full((MM_ROWS, 128), -jnp.inf, jnp.float32)

    x = x_ref[...]
    acc_mn[...] = jnp.minimum(acc_mn[...], x)
    acc_mx[...] = jnp.maximum(acc_mx[...], x)

    @pl.when(i == pl.num_programs(0) - 1)
    def _fin():
        mn_ref[...] = jnp.min(acc_mn[...]).reshape(1, 1)
        mx_ref[...] = jnp.max(acc_mx[...]).reshape(1, 1)


def _assemble_body(idx_ref, dist_ref, ang_ref, mn_ref, mx_ref, out_ref):
    mn = mn_ref[0, 0]
    scale = 1.0 / (mx_ref[0, 0] - mn)
    idxf = idx_ref[...].astype(jnp.float32)
    sel = (lax.broadcasted_iota(jnp.int32, (W, W * NCLS), 0)
           == lax.broadcasted_iota(jnp.int32, (W, W * NCLS), 1) // NCLS
           ).astype(jnp.float32)
    rep = jnp.dot(idxf, sel, preferred_element_type=jnp.float32)
    mod = (lax.broadcasted_iota(jnp.int32, (ROWS_C, W * NCLS), 1)
           % NCLS).astype(jnp.float32)
    onehot = (rep == mod).astype(jnp.float32)
    dn = (dist_ref[...] - mn) * scale
    out_ref[...] = jnp.concatenate([onehot, dn, ang_ref[...]], axis=1)


def kernel(dist, angle, idx_t, index_t, index_h):
    dist = dist.astype(jnp.float32)
    angle = angle.astype(jnp.float32)
    idx_t = idx_t.astype(jnp.int32)
    index_t = index_t.astype(jnp.int32)
    index_h = index_h.astype(jnp.int32)

    rb = 512
    dtab, atab = pl.pallas_call(
        _tables_body,
        grid=(B // rb,),
        in_specs=[
            pl.BlockSpec((rb, L), lambda i: (i, 0)),
            pl.BlockSpec((rb, L, A), lambda i: (i, 0, 0)),
        ],
        out_specs=[
            pl.BlockSpec((rb, T), lambda i: (i, 0)),
            pl.BlockSpec((rb, T, 8), lambda i: (i, 0, 0)),
        ],
        out_shape=[
            jax.ShapeDtypeStruct((B, T), jnp.float32),
            jax.ShapeDtypeStruct((B, T, 8), jnp.float32),
        ],
    )(dist, angle)
    dtab_flat = dtab.reshape(B * T)
    atab_flat = atab.reshape(B * T, 8)

    ib = 1000
    fidx = pl.pallas_call(
        _flatidx_body,
        grid=(H // ib,),
        in_specs=[
            pl.BlockSpec((ib, 1), lambda i: (i, 0)),
            pl.BlockSpec((ib, W), lambda i: (i, 0)),
        ],
        out_specs=pl.BlockSpec((ib, W), lambda i: (i, 0)),
        out_shape=jax.ShapeDtypeStruct((H, W), jnp.int32),
    )(index_h.reshape(H, 1), index_t)
    ff = fidx.reshape(TOT)
    fidx_pad = jnp.concatenate(
        [ff, jnp.broadcast_to(ff[0], (PAD - TOT,))])

    mesh = plsc.VectorSubcoreMesh(core_axis_name="c", subcore_axis_name="s")
    gather = functools.partial(
        pl.kernel,
        mesh=mesh,
        compiler_params=pltpu.CompilerParams(use_tc_tiling_on_sc=False),
        out_type=[
            jax.ShapeDtypeStruct((PAD,), jnp.float32),
            jax.ShapeDtypeStruct((PAD, 8), jnp.float32),
        ],
        scratch_types=[
            pltpu.VMEM((CH,), jnp.int32),
            pltpu.VMEM((CH,), jnp.float32),
            pltpu.VMEM((CH, 8), jnp.float32),
            pltpu.SemaphoreType.DMA,
            pltpu.SemaphoreType.DMA,
        ],
    )(_gather_body)
    dist_g, ang_g = gather(dtab_flat, atab_flat, fidx_pad)

    mn, mx = pl.pallas_call(
        _minmax_body,
        grid=(PAD // 128 // MM_ROWS,),
        in_specs=[pl.BlockSpec((MM_ROWS, 128), lambda i: (i, 0))],
        out_specs=[
            pl.BlockSpec((1, 1), lambda i: (0, 0)),
            pl.BlockSpec((1, 1), lambda i: (0, 0)),
        ],
        out_shape=[
            jax.ShapeDtypeStruct((1, 1), jnp.float32),
            jax.ShapeDtypeStruct((1, 1), jnp.float32),
        ],
        scratch_shapes=[
            pltpu.VMEM((MM_ROWS, 128), jnp.float32),
            pltpu.VMEM((MM_ROWS, 128), jnp.float32),
        ],
    )(dist_g.reshape(PAD // 128, 128))

    dist_t = dist_g[:TOT].reshape(H, W)
    ang_t = ang_g[:TOT].reshape(H, W * 8)

    out = pl.pallas_call(
        _assemble_body,
        grid=(H // ROWS_C,),
        in_specs=[
            pl.BlockSpec((ROWS_C, W), lambda i: (i, 0)),
            pl.BlockSpec((ROWS_C, W), lambda i: (i, 0)),
            pl.BlockSpec((ROWS_C, W * 8), lambda i: (i, 0)),
            pl.BlockSpec((1, 1), lambda i: (0, 0)),
            pl.BlockSpec((1, 1), lambda i: (0, 0)),
        ],
        out_specs=pl.BlockSpec((ROWS_C, W * NCLS + W + W * 8),
                               lambda i: (i, 0)),
        out_shape=jax.ShapeDtypeStruct((H, W * NCLS + W + W * 8),
                                       jnp.float32),
    )(idx_t, dist_t, ang_t, mn, mx)
    return out
